# baseline (speedup 1.0000x reference)
import dataclasses

import jax
import jax.numpy as jnp
from jax import lax
from jax.experimental import pallas as pl
from jax.experimental.pallas import tpu as pltpu
from jax.experimental.pallas import tpu_sc as plsc

B = 16
T = 480000
IN_SIZE = 16000
K = 6000
NBLK = T // K
GRPS = K // 16
CK = 24000


def _body(se_hbm, orig_hbm, gen_hbm, out_hbm,
          se_ref, gen_ref, in_ref, out_ref, big0, big1, sem0, sem1, gen_sem):
    b = lax.axis_index("s")
    h = lax.axis_index("c")
    w = b * 2 + h

    pltpu.sync_copy(se_hbm, se_ref)

    sv = se_ref[pl.ds(0, 16)]
    ev = se_ref[pl.ds(16, 16)]
    lane = lax.iota(jnp.int32, 16)

    bad = jnp.max(jnp.where(ev - sv <= 0, 1, 0)) > 0
    nb = jnp.where(bad, 0, 1)
    sv2 = sv * nb
    ev2 = ev * nb

    sidx_v = sv2 // K
    eidx_v = (ev2 + (K - 1)) // K
    ncc_v = eidx_v - sidx_v
    cum_v = jnp.cumsum(ncc_v)
    exc_v = cum_v - ncc_v
    ncc_total = jnp.max(cum_v)

    bsel = lane == b
    sidx = jnp.sum(jnp.where(bsel, sidx_v, 0))
    eidx = jnp.sum(jnp.where(bsel, eidx_v, 0))
    row0 = b * T
    half0 = h * (NBLK // 2)

    lo1 = half0 * K
    hi1 = jnp.minimum(sidx, half0 + NBLK // 2) * K
    len1 = jnp.maximum(hi1 - lo1, 0)
    lo2 = jnp.maximum(eidx, half0) * K
    hi2 = (half0 + NBLK // 2) * K
    len2 = jnp.maximum(hi2 - lo2, 0)

    def _copy_run(lo, length):
        n = length // CK

        @pl.loop(0, n)
        def _chunk(i):
            off = row0 + lo + i * CK
            even = (i & 1) == 0

            @pl.when(even)
            def _():
                @pl.when(i >= 2)
                def _():
                    pltpu.make_async_copy(orig_hbm.at[pl.ds(0, CK)], big0,
                                          sem0).wait()
                pltpu.sync_copy(orig_hbm.at[pl.ds(off, CK)], big0)
                pltpu.async_copy(big0, out_hbm.at[pl.ds(off, CK)], sem0)

            @pl.when(jnp.logical_not(even))
            def _():
                @pl.when(i >= 2)
                def _():
                    pltpu.make_async_copy(orig_hbm.at[pl.ds(0, CK)], big1,
                                          sem1).wait()
                pltpu.sync_copy(orig_hbm.at[pl.ds(off, CK)], big1)
                pltpu.async_copy(big1, out_hbm.at[pl.ds(off, CK)], sem1)

        @pl.loop(lo + n * CK, lo + length, step=K)
        def _tail(off):
            pltpu.sync_copy(orig_hbm.at[pl.ds(row0 + off, K)], in_ref)
            pltpu.sync_copy(in_ref, out_hbm.at[pl.ds(row0 + off, K)])

        return (n + 1) >> 1, n >> 1

    c0a, c1a = _copy_run(lo1, len1)
    @pl.loop(0, jnp.minimum(c0a, 1))
    def _d0(i):
        pltpu.make_async_copy(orig_hbm.at[pl.ds(0, CK)], big0, sem0).wait()

    @pl.loop(0, jnp.minimum(c1a, 1))
    def _d1(i):
        pltpu.make_async_copy(orig_hbm.at[pl.ds(0, CK)], big1, sem1).wait()

    c0b, c1b = _copy_run(lo2, len2)
    c0 = jnp.minimum(c0b, 1)
    c1 = jnp.minimum(c1b, 1)

    qa = (w * ncc_total) >> 5
    qb = ((w + 1) * ncc_total) >> 5

    r_first = jnp.sum(jnp.where(exc_v <= qa, 1, 0)) - 1

    @pl.when(qa < qb)
    def _():
        pltpu.async_copy(gen_hbm.at[pl.ds(r_first * IN_SIZE, IN_SIZE)],
                         gen_ref, gen_sem)

    zero_v = jnp.zeros((16,), jnp.float32)
    one_v = jnp.ones((16,), jnp.float32)

    @pl.loop(qa, qb, init_carry=jnp.int32(-1))
    def _chunk(q, rprev):
        r = jnp.sum(jnp.where(exc_v <= q, 1, 0)) - 1

        @pl.when(rprev == -1)
        def _():
            pltpu.make_async_copy(gen_hbm.at[pl.ds(0, IN_SIZE)], gen_ref,
                                  gen_sem).wait()

        @pl.when((r != rprev) & (rprev != -1))
        def _():
            pltpu.sync_copy(gen_hbm.at[pl.ds(r * IN_SIZE, IN_SIZE)], gen_ref)

        rsel = lane == r
        start = jnp.sum(jnp.where(rsel, sv2, 0))
        end = jnp.sum(jnp.where(rsel, ev2, 0))
        exc_r = jnp.sum(jnp.where(rsel, exc_v, 0))
        sidx_r = jnp.sum(jnp.where(rsel, sidx_v, 0))
        a = (sidx_r + (q - exc_r)) * K
        dst = r * T + a
        t0 = a - start

        L = end - start
        L_v = jnp.full((16,), L, dtype=jnp.int32)
        scale_v = jnp.float32(IN_SIZE) / L_v.astype(jnp.float32)
        cf = jnp.minimum(1000, L >> 2)

        interior = (a >= start + cf) & (a + K <= end - cf)

        @pl.when(interior)
        def _():
            @pl.loop(0, GRPS, unroll=8)
            def _grp(g):
                tv = jnp.full((16,), t0 + g * 16, jnp.int32) + lane
                tf = tv.astype(jnp.float32)
                src = (tf + 0.5) * scale_v - 0.5
                lo = src.astype(jnp.int32)
                wgt = src - lo.astype(jnp.float32)
                xlo = plsc.load_gather(gen_ref, [lo])
                xhi = plsc.load_gather(gen_ref, [lo + 1])
                out_ref[pl.ds(g * 16, 16)] = xlo * (one_v - wgt) + xhi * wgt

        @pl.when(jnp.logical_not(interior))
        def _():
            pltpu.sync_copy(orig_hbm.at[pl.ds(dst, K)], in_ref)
            cfm1 = jnp.maximum(cf - 1, 1)
            inv_v = (jnp.float32(1.0)
                     / jnp.full((16,), cfm1, jnp.int32).astype(jnp.float32))
            cf_v = jnp.full((16,), cf, dtype=jnp.int32)
            cf1_v = cf_v == 1
            cfpos_v = cf_v > 0
            Lmc_v = jnp.full((16,), L - cf, dtype=jnp.int32)

            @pl.loop(0, GRPS, unroll=4)
            def _grp(g):
                tv = jnp.full((16,), t0 + g * 16, jnp.int32) + lane
                tf = tv.astype(jnp.float32)
                src = (tf + 0.5) * scale_v - 0.5
                src = jnp.minimum(jnp.maximum(src, 0.0),
                                  jnp.float32(IN_SIZE - 1))
                lo = src.astype(jnp.int32)
                wgt = src - lo.astype(jnp.float32)
                hi = jnp.minimum(lo + 1, IN_SIZE - 1)
                xlo = plsc.load_gather(gen_ref, [lo])
                xhi = plsc.load_gather(gen_ref, [hi])
                val = xlo * (one_v - wgt) + xhi * wgt
                fin = jnp.where(cf1_v, zero_v, tf * inv_v)
                kf = (tv - Lmc_v).astype(jnp.float32)
                fout = jnp.where(cf1_v, one_v, one_v - kf * inv_v)
                fade = jnp.where(tv < cf_v, fin,
                                 jnp.where(tv >= Lmc_v, fout, one_v))
                fade = jnp.where(cfpos_v, fade, one_v)
                ingap = (tv >= 0) & (tv < L_v)
                ov = in_ref[pl.ds(g * 16, 16)]
                out_ref[pl.ds(g * 16, 16)] = jnp.where(ingap, val * fade, ov)

        pltpu.sync_copy(out_ref, out_hbm.at[pl.ds(dst, K)])
        return r

    @pl.loop(0, c0)
    def _drain0(i):
        pltpu.make_async_copy(orig_hbm.at[pl.ds(0, CK)], big0, sem0).wait()

    @pl.loop(0, c1)
    def _drain1(i):
        pltpu.make_async_copy(orig_hbm.at[pl.ds(0, CK)], big1, sem1).wait()


@jax.jit
def _combine(original_audio, generated_audio, se):
    cp = pltpu.CompilerParams()
    if "needs_layout_passes" in pltpu.CompilerParams.__dataclass_fields__:
        cp = dataclasses.replace(cp, needs_layout_passes=False)
    kfn = pl.kernel(
        _body,
        out_type=jax.ShapeDtypeStruct((B * T,), jnp.float32),
        compiler_params=cp,
        mesh=plsc.VectorSubcoreMesh(core_axis_name="c", subcore_axis_name="s"),
        scratch_types=[
            pltpu.VMEM((32,), jnp.int32),
            pltpu.VMEM((IN_SIZE,), jnp.float32),
            pltpu.VMEM((K,), jnp.float32),
            pltpu.VMEM((K,), jnp.float32),
            pltpu.VMEM((CK,), jnp.float32),
            pltpu.VMEM((CK,), jnp.float32),
            pltpu.SemaphoreType.DMA,
            pltpu.SemaphoreType.DMA,
            pltpu.SemaphoreType.DMA,
        ],
    )
    flat = kfn(se, original_audio.reshape(B * T),
               generated_audio.reshape(B * IN_SIZE))
    return flat.reshape(B, T)


def kernel(original_audio, generated_audio, gaps):
    starts = gaps[:, 0, 0].astype(jnp.int32)
    ends = gaps[:, 0, 1].astype(jnp.int32)
    se = jnp.concatenate([starts, ends])
    return _combine(original_audio, generated_audio, se)


# speedup vs baseline: 1.0061x; 1.0061x over previous
import dataclasses

import jax
import jax.numpy as jnp
from jax import lax
from jax.experimental import pallas as pl
from jax.experimental.pallas import tpu as pltpu
from jax.experimental.pallas import tpu_sc as plsc

B = 16
T = 480000
IN_SIZE = 16000
K = 6000
NBLK = T // K
GRPS = K // 16
CK = 24000


def _body(se_hbm, orig_hbm, gen_hbm, out_hbm,
          se_ref, gen_ref, in_ref, out_ref, big0, big1, co0, co1,
          sem0, sem1, gen_sem, semA, semB):
    b = lax.axis_index("s")
    h = lax.axis_index("c")
    w = b * 2 + h

    pltpu.sync_copy(se_hbm, se_ref)

    sv = se_ref[pl.ds(0, 16)]
    ev = se_ref[pl.ds(16, 16)]
    lane = lax.iota(jnp.int32, 16)

    bad = jnp.max(jnp.where(ev - sv <= 0, 1, 0)) > 0
    nb = jnp.where(bad, 0, 1)
    sv2 = sv * nb
    ev2 = ev * nb

    sidx_v = sv2 // K
    eidx_v = (ev2 + (K - 1)) // K
    ncc_v = eidx_v - sidx_v
    cum_v = jnp.cumsum(ncc_v)
    exc_v = cum_v - ncc_v
    ncc_total = jnp.max(cum_v)

    bsel = lane == b
    sidx = jnp.sum(jnp.where(bsel, sidx_v, 0))
    eidx = jnp.sum(jnp.where(bsel, eidx_v, 0))
    row0 = b * T
    half0 = h * (NBLK // 2)

    lo1 = half0 * K
    hi1 = jnp.minimum(sidx, half0 + NBLK // 2) * K
    len1 = jnp.maximum(hi1 - lo1, 0)
    lo2 = jnp.maximum(eidx, half0) * K
    hi2 = (half0 + NBLK // 2) * K
    len2 = jnp.maximum(hi2 - lo2, 0)

    def _copy_run(lo, length):
        n = length // CK

        @pl.loop(0, n)
        def _chunk(i):
            off = row0 + lo + i * CK
            even = (i & 1) == 0

            @pl.when(even)
            def _():
                @pl.when(i >= 2)
                def _():
                    pltpu.make_async_copy(orig_hbm.at[pl.ds(0, CK)], big0,
                                          sem0).wait()
                pltpu.sync_copy(orig_hbm.at[pl.ds(off, CK)], big0)
                pltpu.async_copy(big0, out_hbm.at[pl.ds(off, CK)], sem0)

            @pl.when(jnp.logical_not(even))
            def _():
                @pl.when(i >= 2)
                def _():
                    pltpu.make_async_copy(orig_hbm.at[pl.ds(0, CK)], big1,
                                          sem1).wait()
                pltpu.sync_copy(orig_hbm.at[pl.ds(off, CK)], big1)
                pltpu.async_copy(big1, out_hbm.at[pl.ds(off, CK)], sem1)

        @pl.loop(lo + n * CK, lo + length, step=K)
        def _tail(off):
            pltpu.sync_copy(orig_hbm.at[pl.ds(row0 + off, K)], in_ref)
            pltpu.sync_copy(in_ref, out_hbm.at[pl.ds(row0 + off, K)])

        return (n + 1) >> 1, n >> 1

    c0a, c1a = _copy_run(lo1, len1)
    @pl.loop(0, jnp.minimum(c0a, 1))
    def _d0(i):
        pltpu.make_async_copy(orig_hbm.at[pl.ds(0, CK)], big0, sem0).wait()

    @pl.loop(0, jnp.minimum(c1a, 1))
    def _d1(i):
        pltpu.make_async_copy(orig_hbm.at[pl.ds(0, CK)], big1, sem1).wait()

    c0b, c1b = _copy_run(lo2, len2)
    c0 = jnp.minimum(c0b, 1)
    c1 = jnp.minimum(c1b, 1)

    qa = (w * ncc_total) >> 5
    qb = ((w + 1) * ncc_total) >> 5

    r_first = jnp.sum(jnp.where(exc_v <= qa, 1, 0)) - 1

    @pl.when(qa < qb)
    def _():
        pltpu.async_copy(gen_hbm.at[pl.ds(r_first * IN_SIZE, IN_SIZE)],
                         gen_ref, gen_sem)

    zero_v = jnp.zeros((16,), jnp.float32)
    one_v = jnp.ones((16,), jnp.float32)

    @pl.loop(qa, qb,
             init_carry=(jnp.int32(-1), jnp.int32(0), jnp.int32(0)))
    def _fincarry(q, carry):
        rprev, f0, f1 = carry
        r = jnp.sum(jnp.where(exc_v <= q, 1, 0)) - 1

        @pl.when(rprev == -1)
        def _():
            pltpu.make_async_copy(gen_hbm.at[pl.ds(0, IN_SIZE)], gen_ref,
                                  gen_sem).wait()

        @pl.when((r != rprev) & (rprev != -1))
        def _():
            pltpu.sync_copy(gen_hbm.at[pl.ds(r * IN_SIZE, IN_SIZE)], gen_ref)

        rsel = lane == r
        start = jnp.sum(jnp.where(rsel, sv2, 0))
        end = jnp.sum(jnp.where(rsel, ev2, 0))
        exc_r = jnp.sum(jnp.where(rsel, exc_v, 0))
        sidx_r = jnp.sum(jnp.where(rsel, sidx_v, 0))
        a = (sidx_r + (q - exc_r)) * K
        dst = r * T + a
        t0 = a - start

        L = end - start
        L_v = jnp.full((16,), L, dtype=jnp.int32)
        scale_v = jnp.float32(IN_SIZE) / L_v.astype(jnp.float32)
        cf = jnp.minimum(1000, L >> 2)

        interior = (a >= start + cf) & (a + K <= end - cf)
        p = (q - qa) & 1

        def _fast(buf, sem, fired):
            @pl.when(fired == 1)
            def _():
                pltpu.make_async_copy(orig_hbm.at[pl.ds(0, K)], buf,
                                      sem).wait()

            @pl.loop(0, GRPS, unroll=8)
            def _grp(g):
                tv = jnp.full((16,), t0 + g * 16, jnp.int32) + lane
                tf = tv.astype(jnp.float32)
                src = (tf + 0.5) * scale_v - 0.5
                lo = src.astype(jnp.int32)
                wgt = src - lo.astype(jnp.float32)
                xlo = plsc.load_gather(gen_ref, [lo])
                xhi = plsc.load_gather(gen_ref, [lo + 1])
                buf[pl.ds(g * 16, 16)] = xlo * (one_v - wgt) + xhi * wgt

            pltpu.async_copy(buf, out_hbm.at[pl.ds(dst, K)], sem)

        @pl.when(interior & (p == 0))
        def _():
            _fast(co0, semA, f0)

        @pl.when(interior & (p == 1))
        def _():
            _fast(co1, semB, f1)

        @pl.when(jnp.logical_not(interior))
        def _():
            pltpu.sync_copy(orig_hbm.at[pl.ds(dst, K)], in_ref)
            cfm1 = jnp.maximum(cf - 1, 1)
            inv_v = (jnp.float32(1.0)
                     / jnp.full((16,), cfm1, jnp.int32).astype(jnp.float32))
            cf_v = jnp.full((16,), cf, dtype=jnp.int32)
            cf1_v = cf_v == 1
            cfpos_v = cf_v > 0
            Lmc_v = jnp.full((16,), L - cf, dtype=jnp.int32)

            @pl.loop(0, GRPS, unroll=4)
            def _grp(g):
                tv = jnp.full((16,), t0 + g * 16, jnp.int32) + lane
                tf = tv.astype(jnp.float32)
                src = (tf + 0.5) * scale_v - 0.5
                src = jnp.minimum(jnp.maximum(src, 0.0),
                                  jnp.float32(IN_SIZE - 1))
                lo = src.astype(jnp.int32)
                wgt = src - lo.astype(jnp.float32)
                hi = jnp.minimum(lo + 1, IN_SIZE - 1)
                xlo = plsc.load_gather(gen_ref, [lo])
                xhi = plsc.load_gather(gen_ref, [hi])
                val = xlo * (one_v - wgt) + xhi * wgt
                fin = jnp.where(cf1_v, zero_v, tf * inv_v)
                kf = (tv - Lmc_v).astype(jnp.float32)
                fout = jnp.where(cf1_v, one_v, one_v - kf * inv_v)
                fade = jnp.where(tv < cf_v, fin,
                                 jnp.where(tv >= Lmc_v, fout, one_v))
                fade = jnp.where(cfpos_v, fade, one_v)
                ingap = (tv >= 0) & (tv < L_v)
                ov = in_ref[pl.ds(g * 16, 16)]
                out_ref[pl.ds(g * 16, 16)] = jnp.where(ingap, val * fade, ov)

            pltpu.sync_copy(out_ref, out_hbm.at[pl.ds(dst, K)])

        f0n = jnp.where(interior & (p == 0), 1, f0)
        f1n = jnp.where(interior & (p == 1), 1, f1)
        return (r, f0n, f1n)

    _, ff0, ff1 = _fincarry

    @pl.loop(0, ff0)
    def _dc0(i):
        pltpu.make_async_copy(orig_hbm.at[pl.ds(0, K)], co0, semA).wait()

    @pl.loop(0, ff1)
    def _dc1(i):
        pltpu.make_async_copy(orig_hbm.at[pl.ds(0, K)], co1, semB).wait()

    @pl.loop(0, c0)
    def _drain0(i):
        pltpu.make_async_copy(orig_hbm.at[pl.ds(0, CK)], big0, sem0).wait()

    @pl.loop(0, c1)
    def _drain1(i):
        pltpu.make_async_copy(orig_hbm.at[pl.ds(0, CK)], big1, sem1).wait()


@jax.jit
def _combine(original_audio, generated_audio, se):
    cp = pltpu.CompilerParams()
    if "needs_layout_passes" in pltpu.CompilerParams.__dataclass_fields__:
        cp = dataclasses.replace(cp, needs_layout_passes=False)
    kfn = pl.kernel(
        _body,
        out_type=jax.ShapeDtypeStruct((B * T,), jnp.float32),
        compiler_params=cp,
        mesh=plsc.VectorSubcoreMesh(core_axis_name="c", subcore_axis_name="s"),
        scratch_types=[
            pltpu.VMEM((32,), jnp.int32),
            pltpu.VMEM((IN_SIZE,), jnp.float32),
            pltpu.VMEM((K,), jnp.float32),
            pltpu.VMEM((K,), jnp.float32),
            pltpu.VMEM((CK,), jnp.float32),
            pltpu.VMEM((CK,), jnp.float32),
            pltpu.VMEM((K,), jnp.float32),
            pltpu.VMEM((K,), jnp.float32),
            pltpu.SemaphoreType.DMA,
            pltpu.SemaphoreType.DMA,
            pltpu.SemaphoreType.DMA,
            pltpu.SemaphoreType.DMA,
            pltpu.SemaphoreType.DMA,
        ],
    )
    flat = kfn(se, original_audio.reshape(B * T),
               generated_audio.reshape(B * IN_SIZE))
    return flat.reshape(B, T)


def kernel(original_audio, generated_audio, gaps):
    starts = gaps[:, 0, 0].astype(jnp.int32)
    ends = gaps[:, 0, 1].astype(jnp.int32)
    se = jnp.concatenate([starts, ends])
    return _combine(original_audio, generated_audio, se)


# speedup vs baseline: 1.0303x; 1.0241x over previous
import dataclasses

import jax
import jax.numpy as jnp
from jax import lax
from jax.experimental import pallas as pl
from jax.experimental.pallas import tpu as pltpu
from jax.experimental.pallas import tpu_sc as plsc

B = 16
T = 480000
IN_SIZE = 16000
K = 6000
NBLK = T // K
GRPS = K // 16
CK = 24000


def _body(se_hbm, orig_hbm, gen_hbm, out_hbm,
          se_ref, gen_ref, in_ref, out_ref, big0, big1, co0, co1,
          sem0, sem1, gen_sem, semA, semB, semI0, semI1):
    b = lax.axis_index("s")
    h = lax.axis_index("c")
    w = b * 2 + h

    pltpu.sync_copy(se_hbm, se_ref)

    sv = se_ref[pl.ds(0, 16)]
    ev = se_ref[pl.ds(16, 16)]
    lane = lax.iota(jnp.int32, 16)

    bad = jnp.max(jnp.where(ev - sv <= 0, 1, 0)) > 0
    nb = jnp.where(bad, 0, 1)
    sv2 = sv * nb
    ev2 = ev * nb

    sidx_v = sv2 // K
    eidx_v = (ev2 + (K - 1)) // K
    ncc_v = eidx_v - sidx_v
    cum_v = jnp.cumsum(ncc_v)
    exc_v = cum_v - ncc_v
    ncc_total = jnp.max(cum_v)

    bsel = lane == b
    sidx = jnp.sum(jnp.where(bsel, sidx_v, 0))
    eidx = jnp.sum(jnp.where(bsel, eidx_v, 0))
    row0 = b * T
    half0 = h * (NBLK // 2)

    lo1 = half0 * K
    hi1 = jnp.minimum(sidx, half0 + NBLK // 2) * K
    len1 = jnp.maximum(hi1 - lo1, 0)
    lo2 = jnp.maximum(eidx, half0) * K
    hi2 = (half0 + NBLK // 2) * K
    len2 = jnp.maximum(hi2 - lo2, 0)

    def _copy_run(lo, length):
        n = length // CK

        @pl.when(n > 0)
        def _():
            pltpu.async_copy(orig_hbm.at[pl.ds(row0 + lo, CK)], big0, semI0)

        @pl.loop(0, n)
        def _chunk(i):
            off = row0 + lo + i * CK
            even = (i & 1) == 0

            @pl.when(even)
            def _():
                @pl.when(i + 1 < n)
                def _():
                    @pl.when(i >= 1)
                    def _():
                        pltpu.make_async_copy(orig_hbm.at[pl.ds(0, CK)], big1,
                                              sem1).wait()
                    pltpu.async_copy(orig_hbm.at[pl.ds(off + CK, CK)], big1,
                                     semI1)
                pltpu.make_async_copy(orig_hbm.at[pl.ds(off, CK)], big0,
                                      semI0).wait()
                pltpu.async_copy(big0, out_hbm.at[pl.ds(off, CK)], sem0)

            @pl.when(jnp.logical_not(even))
            def _():
                @pl.when(i + 1 < n)
                def _():
                    pltpu.make_async_copy(orig_hbm.at[pl.ds(0, CK)], big0,
                                          sem0).wait()
                    pltpu.async_copy(orig_hbm.at[pl.ds(off + CK, CK)], big0,
                                     semI0)
                pltpu.make_async_copy(orig_hbm.at[pl.ds(0, CK)], big1,
                                      semI1).wait()
                pltpu.async_copy(big1, out_hbm.at[pl.ds(off, CK)], sem1)

        @pl.loop(lo + n * CK, lo + length, step=K)
        def _tail(off):
            pltpu.sync_copy(orig_hbm.at[pl.ds(row0 + off, K)], in_ref)
            pltpu.sync_copy(in_ref, out_hbm.at[pl.ds(row0 + off, K)])

        return (n + 1) >> 1, n >> 1

    c0a, c1a = _copy_run(lo1, len1)
    @pl.loop(0, jnp.minimum(c0a, 1))
    def _d0(i):
        pltpu.make_async_copy(orig_hbm.at[pl.ds(0, CK)], big0, sem0).wait()

    @pl.loop(0, jnp.minimum(c1a, 1))
    def _d1(i):
        pltpu.make_async_copy(orig_hbm.at[pl.ds(0, CK)], big1, sem1).wait()

    c0b, c1b = _copy_run(lo2, len2)
    c0 = jnp.minimum(c0b, 1)
    c1 = jnp.minimum(c1b, 1)

    qa = (w * ncc_total) >> 5
    qb = ((w + 1) * ncc_total) >> 5

    r_first = jnp.sum(jnp.where(exc_v <= qa, 1, 0)) - 1

    @pl.when(qa < qb)
    def _():
        pltpu.async_copy(gen_hbm.at[pl.ds(r_first * IN_SIZE, IN_SIZE)],
                         gen_ref, gen_sem)

    zero_v = jnp.zeros((16,), jnp.float32)
    one_v = jnp.ones((16,), jnp.float32)

    @pl.loop(qa, qb,
             init_carry=(jnp.int32(-1), jnp.int32(0), jnp.int32(0)))
    def _fincarry(q, carry):
        rprev, f0, f1 = carry
        r = jnp.sum(jnp.where(exc_v <= q, 1, 0)) - 1

        @pl.when(rprev == -1)
        def _():
            pltpu.make_async_copy(gen_hbm.at[pl.ds(0, IN_SIZE)], gen_ref,
                                  gen_sem).wait()

        @pl.when((r != rprev) & (rprev != -1))
        def _():
            pltpu.sync_copy(gen_hbm.at[pl.ds(r * IN_SIZE, IN_SIZE)], gen_ref)

        rsel = lane == r
        start = jnp.sum(jnp.where(rsel, sv2, 0))
        end = jnp.sum(jnp.where(rsel, ev2, 0))
        exc_r = jnp.sum(jnp.where(rsel, exc_v, 0))
        sidx_r = jnp.sum(jnp.where(rsel, sidx_v, 0))
        a = (sidx_r + (q - exc_r)) * K
        dst = r * T + a
        t0 = a - start

        L = end - start
        L_v = jnp.full((16,), L, dtype=jnp.int32)
        scale_v = jnp.float32(IN_SIZE) / L_v.astype(jnp.float32)
        cf = jnp.minimum(1000, L >> 2)

        interior = (a >= start + cf) & (a + K <= end - cf)
        p = (q - qa) & 1

        def _fast(buf, sem, fired):
            @pl.when(fired == 1)
            def _():
                pltpu.make_async_copy(orig_hbm.at[pl.ds(0, K)], buf,
                                      sem).wait()

            @pl.loop(0, GRPS, unroll=8)
            def _grp(g):
                tv = jnp.full((16,), t0 + g * 16, jnp.int32) + lane
                tf = tv.astype(jnp.float32)
                src = (tf + 0.5) * scale_v - 0.5
                lo = src.astype(jnp.int32)
                wgt = src - lo.astype(jnp.float32)
                xlo = plsc.load_gather(gen_ref, [lo])
                xhi = plsc.load_gather(gen_ref, [lo + 1])
                buf[pl.ds(g * 16, 16)] = xlo * (one_v - wgt) + xhi * wgt

            pltpu.async_copy(buf, out_hbm.at[pl.ds(dst, K)], sem)

        @pl.when(interior & (p == 0))
        def _():
            _fast(co0, semA, f0)

        @pl.when(interior & (p == 1))
        def _():
            _fast(co1, semB, f1)

        @pl.when(jnp.logical_not(interior))
        def _():
            pltpu.sync_copy(orig_hbm.at[pl.ds(dst, K)], in_ref)
            cfm1 = jnp.maximum(cf - 1, 1)
            inv_v = (jnp.float32(1.0)
                     / jnp.full((16,), cfm1, jnp.int32).astype(jnp.float32))
            cf_v = jnp.full((16,), cf, dtype=jnp.int32)
            cf1_v = cf_v == 1
            cfpos_v = cf_v > 0
            Lmc_v = jnp.full((16,), L - cf, dtype=jnp.int32)

            @pl.loop(0, GRPS, unroll=4)
            def _grp(g):
                tv = jnp.full((16,), t0 + g * 16, jnp.int32) + lane
                tf = tv.astype(jnp.float32)
                src = (tf + 0.5) * scale_v - 0.5
                src = jnp.minimum(jnp.maximum(src, 0.0),
                                  jnp.float32(IN_SIZE - 1))
                lo = src.astype(jnp.int32)
                wgt = src - lo.astype(jnp.float32)
                hi = jnp.minimum(lo + 1, IN_SIZE - 1)
                xlo = plsc.load_gather(gen_ref, [lo])
                xhi = plsc.load_gather(gen_ref, [hi])
                val = xlo * (one_v - wgt) + xhi * wgt
                fin = jnp.where(cf1_v, zero_v, tf * inv_v)
                kf = (tv - Lmc_v).astype(jnp.float32)
                fout = jnp.where(cf1_v, one_v, one_v - kf * inv_v)
                fade = jnp.where(tv < cf_v, fin,
                                 jnp.where(tv >= Lmc_v, fout, one_v))
                fade = jnp.where(cfpos_v, fade, one_v)
                ingap = (tv >= 0) & (tv < L_v)
                ov = in_ref[pl.ds(g * 16, 16)]
                out_ref[pl.ds(g * 16, 16)] = jnp.where(ingap, val * fade, ov)

            pltpu.sync_copy(out_ref, out_hbm.at[pl.ds(dst, K)])

        f0n = jnp.where(interior & (p == 0), 1, f0)
        f1n = jnp.where(interior & (p == 1), 1, f1)
        return (r, f0n, f1n)

    _, ff0, ff1 = _fincarry

    @pl.loop(0, ff0)
    def _dc0(i):
        pltpu.make_async_copy(orig_hbm.at[pl.ds(0, K)], co0, semA).wait()

    @pl.loop(0, ff1)
    def _dc1(i):
        pltpu.make_async_copy(orig_hbm.at[pl.ds(0, K)], co1, semB).wait()

    @pl.loop(0, c0)
    def _drain0(i):
        pltpu.make_async_copy(orig_hbm.at[pl.ds(0, CK)], big0, sem0).wait()

    @pl.loop(0, c1)
    def _drain1(i):
        pltpu.make_async_copy(orig_hbm.at[pl.ds(0, CK)], big1, sem1).wait()


@jax.jit
def _combine(original_audio, generated_audio, se):
    cp = pltpu.CompilerParams()
    if "needs_layout_passes" in pltpu.CompilerParams.__dataclass_fields__:
        cp = dataclasses.replace(cp, needs_layout_passes=False)
    kfn = pl.kernel(
        _body,
        out_type=jax.ShapeDtypeStruct((B * T,), jnp.float32),
        compiler_params=cp,
        mesh=plsc.VectorSubcoreMesh(core_axis_name="c", subcore_axis_name="s"),
        scratch_types=[
            pltpu.VMEM((32,), jnp.int32),
            pltpu.VMEM((IN_SIZE,), jnp.float32),
            pltpu.VMEM((K,), jnp.float32),
            pltpu.VMEM((K,), jnp.float32),
            pltpu.VMEM((CK,), jnp.float32),
            pltpu.VMEM((CK,), jnp.float32),
            pltpu.VMEM((K,), jnp.float32),
            pltpu.VMEM((K,), jnp.float32),
            pltpu.SemaphoreType.DMA,
            pltpu.SemaphoreType.DMA,
            pltpu.SemaphoreType.DMA,
            pltpu.SemaphoreType.DMA,
            pltpu.SemaphoreType.DMA,
            pltpu.SemaphoreType.DMA,
            pltpu.SemaphoreType.DMA,
        ],
    )
    flat = kfn(se, original_audio.reshape(B * T),
               generated_audio.reshape(B * IN_SIZE))
    return flat.reshape(B, T)


def kernel(original_audio, generated_audio, gaps):
    starts = gaps[:, 0, 0].astype(jnp.int32)
    ends = gaps[:, 0, 1].astype(jnp.int32)
    se = jnp.concatenate([starts, ends])
    return _combine(original_audio, generated_audio, se)
